# baseline (speedup 1.0000x reference)
import functools

import jax
import jax.numpy as jnp
from jax import lax
from jax.experimental import pallas as pl
from jax.experimental.pallas import tpu as pltpu
from jax.experimental.pallas import tpu_sc as plsc

N_ROWS = 50000
N_EDGES_TOTAL = 800000
DIM = 64
HDIM = DIM // 2
BATCH_SIZE = 4096

NUM_SC = 2
NUM_TILES = 16
CHUNK = 128
BLK = 5
N_IDX_ROWS = N_EDGES_TOTAL // CHUNK
N_BLOCKS = N_IDX_ROWS // BLK
ZERO_CHUNK = 125
TILE_ACC = N_ROWS // NUM_TILES
WB_CHUNK = 40
N_WB = N_ROWS // WB_CHUNK
GB = BATCH_SIZE // NUM_TILES

_SC_MESH = plsc.VectorSubcoreMesh(core_axis_name="c", subcore_axis_name="s")
_SC_PARAMS = pltpu.CompilerParams(use_tc_tiling_on_sc=False)


def _xform_planes(idx2d, c):
    n, m = idx2d.shape
    for k in range(n):
        for i in range(m // 16):
            v = idx2d[k, pl.ds(i * 16, 16)]
            idx2d[k, pl.ds(i * 16, 16)] = v * 2 + c


def _xform_planes_1d(idx1d, c):
    for i in range(idx1d.shape[0] // 16):
        v = idx1d[pl.ds(i * 16, 16)]
        idx1d[pl.ds(i * 16, 16)] = v * 2 + c


def _spmm_body(mode, *refs):
    first = mode == "first"
    tail = mode == "tail"
    it = iter(refs)
    src_hbm, dst_hbm, tab_flat = next(it), next(it), next(it)
    if tail:
        t2_flat, rs_in, u_idx, i_idx = next(it), next(it), next(it), next(it)
        u_out, i_out, rsg_out = next(it), next(it), next(it)
        out_hbm = None
    else:
        out_hbm = next(it)
    if first:
        rs_out = next(it)
    idx_s, idx_d, rows = next(it), next(it), next(it)
    if first:
        ones, acc, acc1, sem_g, sem_s, sem_1 = it
    elif tail:
        bidx, bval, acc, sem_g, sem_s = it
    else:
        acc, sem_g, sem_s = it

    c = lax.axis_index("c")
    s = lax.axis_index("s")

    def zrow(r, _):
        for j in range(HDIM // 16):
            rows[0, r, pl.ds(j * 16, 16)] = jnp.zeros((16,), jnp.float32)
        return 0

    lax.fori_loop(0, CHUNK, zrow, 0)
    if first:
        for j in range(CHUNK // 16):
            ones[pl.ds(j * 16, 16)] = jnp.ones((16,), jnp.float32)

    def zbody(j, _):
        pltpu.sync_copy(
            rows.at[0, pl.ds(0, ZERO_CHUNK), :],
            acc.at[pl.ds(s * TILE_ACC + j * ZERO_CHUNK, ZERO_CHUNK), :],
        )
        return 0

    lax.fori_loop(0, TILE_ACC // ZERO_CHUNK, zbody, 0)
    if first:
        @pl.when(c == 0)
        def _():
            zcol = rows.at[0, 0, pl.ds(0, 16)]
            n_z1_chunks = N_ROWS // 16
            n_z1 = n_z1_chunks // NUM_TILES + jnp.where(
                s < n_z1_chunks % NUM_TILES, 1, 0)

            def z1body(j, _):
                pltpu.sync_copy(zcol, acc1.at[pl.ds((s + j * NUM_TILES) * 16, 16)])
                return 0

            lax.fori_loop(0, n_z1, z1body, 0)

    plsc.subcore_barrier()

    n_my = N_BLOCKS // NUM_TILES + jnp.where(s < N_BLOCKS % NUM_TILES, 1, 0)

    def ebody(j, _):
        @pl.when(j > 0)
        def _():
            for k in range(BLK):
                pltpu.make_async_copy(tab_flat.at[pl.ds(0, CHUNK), :],
                                      rows.at[k], sem_s).wait()
            if first:
                @pl.when(c == 0)
                def _():
                    for k in range(BLK):
                        pltpu.make_async_copy(rs_out.at[pl.ds(0, CHUNK)],
                                              ones, sem_1).wait()

        b = (s + j * NUM_TILES) * BLK
        pltpu.sync_copy(src_hbm.at[pl.ds(b, BLK), :], idx_s)
        pltpu.sync_copy(dst_hbm.at[pl.ds(b, BLK), :], idx_d)
        _xform_planes(idx_s, c)
        gathers = [
            pltpu.async_copy(tab_flat.at[idx_s.at[k]], rows.at[k], sem_g)
            for k in range(BLK)
        ]
        for k in range(BLK):
            gathers[k].wait()
            pltpu.async_copy(rows.at[k], acc.at[idx_d.at[k]], sem_s, add=True)
            if first:
                @pl.when(c == 0)
                def _():
                    pltpu.async_copy(ones, acc1.at[idx_d.at[k]], sem_1, add=True)
        return 0

    lax.fori_loop(0, n_my, ebody, 0)
    for k in range(BLK):
        pltpu.make_async_copy(tab_flat.at[pl.ds(0, CHUNK), :], rows.at[k],
                              sem_s).wait()
    if first:
        @pl.when(c == 0)
        def _():
            for k in range(BLK):
                pltpu.make_async_copy(rs_out.at[pl.ds(0, CHUNK)], ones,
                                      sem_1).wait()
    plsc.subcore_barrier()

    if not tail:
        n_wb = N_WB // NUM_TILES + jnp.where(s < N_WB % NUM_TILES, 1, 0)

        def wbody(j, _):
            r0 = (s + j * NUM_TILES) * WB_CHUNK
            pltpu.sync_copy(acc.at[pl.ds(r0, WB_CHUNK), :],
                            out_hbm.at[pl.ds(r0, WB_CHUNK), c, :])
            if first:
                @pl.when(c == 0)
                def _():
                    pltpu.sync_copy(acc1.at[pl.ds(r0, WB_CHUNK)],
                                    rs_out.at[pl.ds(r0, WB_CHUNK)])
            return 0

        lax.fori_loop(0, n_wb, wbody, 0)

    if tail:
        base = s * GB
        for half in range(GB // CHUNK):
            o = base + half * CHUNK
            pltpu.sync_copy(u_idx.at[pl.ds(o, CHUNK)], bidx)

            @pl.when(c == 0)
            def _():
                pltpu.async_copy(rs_in.at[bidx], bval, sem_g).wait()
                pltpu.sync_copy(bval, rsg_out.at[pl.ds(o, CHUNK)])

            _xform_planes_1d(bidx, c)
            pltpu.async_copy(t2_flat.at[bidx], rows.at[0], sem_g).wait()
            pltpu.sync_copy(rows.at[0], u_out.at[pl.ds(o, CHUNK), c, :])

            pltpu.sync_copy(i_idx.at[pl.ds(o, CHUNK)], bidx)
            pltpu.sync_copy(acc.at[bidx], rows.at[0])
            pltpu.sync_copy(rows.at[0], i_out.at[pl.ds(o, CHUNK), c, :])


def _make_spmm(mode):
    in_ty = None
    out_type = [jax.ShapeDtypeStruct((N_ROWS, NUM_SC, HDIM), jnp.float32)]
    scratch = [
        pltpu.VMEM((BLK, CHUNK), jnp.int32),
        pltpu.VMEM((BLK, CHUNK), jnp.int32),
        pltpu.VMEM((BLK, CHUNK, HDIM), jnp.float32),
    ]
    if mode == "first":
        out_type.append(jax.ShapeDtypeStruct((N_ROWS,), jnp.float32))
        scratch += [
            pltpu.VMEM((CHUNK,), jnp.float32),
            pltpu.VMEM_SHARED((N_ROWS, HDIM), jnp.float32),
            pltpu.VMEM_SHARED((N_ROWS,), jnp.float32),
        ]
        scratch += [pltpu.SemaphoreType.DMA] * 3
    elif mode == "tail":
        out_type = [
            jax.ShapeDtypeStruct((BATCH_SIZE, NUM_SC, HDIM), jnp.float32),
            jax.ShapeDtypeStruct((BATCH_SIZE, NUM_SC, HDIM), jnp.float32),
            jax.ShapeDtypeStruct((BATCH_SIZE,), jnp.float32),
        ]
        scratch += [
            pltpu.VMEM((CHUNK,), jnp.int32),
            pltpu.VMEM((CHUNK,), jnp.float32),
            pltpu.VMEM_SHARED((N_ROWS, HDIM), jnp.float32),
        ]
        scratch += [pltpu.SemaphoreType.DMA] * 2
    else:
        scratch += [pltpu.VMEM_SHARED((N_ROWS, HDIM), jnp.float32)]
        scratch += [pltpu.SemaphoreType.DMA] * 2
    return pl.kernel(
        functools.partial(_spmm_body, mode),
        out_type=out_type[0] if len(out_type) == 1 else tuple(out_type),
        mesh=_SC_MESH,
        scratch_types=scratch,
        compiler_params=_SC_PARAMS,
        name="spmm_" + mode,
    )


_spmm_first = _make_spmm("first")
_spmm = _make_spmm("mid")
_spmm_tail = _make_spmm("tail")


_SCALE_BLK = 2000


def _scale_kernel(t_ref, rs_ref, o_ref):
    rs = rs_ref[...]
    inv = jnp.where(rs > 0, 1.0 / rs, 0.0)
    o_ref[...] = t_ref[...] * (inv * inv)


def _scale_rows(t, rs):
    grid = N_ROWS // _SCALE_BLK
    return pl.pallas_call(
        _scale_kernel,
        grid=(grid,),
        in_specs=[
            pl.BlockSpec((_SCALE_BLK, DIM), lambda i: (i, 0)),
            pl.BlockSpec((_SCALE_BLK, 1), lambda i: (i, 0)),
        ],
        out_specs=pl.BlockSpec((_SCALE_BLK, DIM), lambda i: (i, 0)),
        out_shape=jax.ShapeDtypeStruct((N_ROWS, DIM), jnp.float32),
    )(t, rs.reshape(N_ROWS, 1))


def _score_kernel(u_ref, i_ref, rs_ref, o_ref):
    dot = jnp.sum(u_ref[...] * i_ref[...], axis=1, keepdims=True)
    rs = rs_ref[...]
    inv = jnp.where(rs > 0, 1.0 / rs, 0.0)
    o_ref[...] = jax.nn.sigmoid(dot * inv)


def _score(u_rows, i_rows, rs_g):
    out = pl.pallas_call(
        _score_kernel,
        out_shape=jax.ShapeDtypeStruct((BATCH_SIZE, 1), jnp.float32),
    )(u_rows, i_rows, rs_g.reshape(BATCH_SIZE, 1))
    return out.reshape(BATCH_SIZE)


def kernel(user_indices, item_indices, user_table, item_table, adj_rows, adj_cols):
    del user_table
    src_r = adj_rows.reshape(N_IDX_ROWS, CHUNK)
    src_c = adj_cols.reshape(N_IDX_ROWS, CHUNK)

    def flat(t):
        return t.reshape(NUM_SC * N_ROWS, HDIM)

    t1, rowsum = _spmm_first(src_c, src_r, flat(item_table))
    us1 = _scale_rows(t1.reshape(N_ROWS, DIM), rowsum)
    i1 = _spmm(src_r, src_c, flat(us1))
    t2 = _spmm(src_c, src_r, flat(i1.reshape(N_ROWS, DIM)))
    us2 = _scale_rows(t2.reshape(N_ROWS, DIM), rowsum)
    u_rows, i_rows, rs_g = _spmm_tail(
        src_r, src_c, flat(us2),
        t2.reshape(NUM_SC * N_ROWS, HDIM), rowsum, user_indices, item_indices)
    return _score(u_rows.reshape(BATCH_SIZE, DIM),
                  i_rows.reshape(BATCH_SIZE, DIM), rs_g)


# speedup vs baseline: 1.3041x; 1.3041x over previous
import functools

import jax
import jax.numpy as jnp
from jax import lax
from jax.experimental import pallas as pl
from jax.experimental.pallas import tpu as pltpu
from jax.experimental.pallas import tpu_sc as plsc

N_ROWS = 50000
N_EDGES_TOTAL = 800000
DIM = 64
HDIM = DIM // 2
BATCH_SIZE = 4096

NUM_SC = 2
NUM_TILES = 16
CHUNK = 128
BLK = 5
N_IDX_ROWS = N_EDGES_TOTAL // CHUNK
N_BLOCKS = N_IDX_ROWS // BLK
ZERO_CHUNK = 125
TILE_ACC = N_ROWS // NUM_TILES
WB_CHUNK = 400
N_WB = N_ROWS // WB_CHUNK
GB = BATCH_SIZE // NUM_TILES

_SC_MESH = plsc.VectorSubcoreMesh(core_axis_name="c", subcore_axis_name="s")
_SC_PARAMS = pltpu.CompilerParams(use_tc_tiling_on_sc=False)


def _xform_planes(idx2d, mult, off):
    n, m = idx2d.shape
    for k in range(n):
        for i in range(m // 16):
            v = idx2d[k, pl.ds(i * 16, 16)]
            idx2d[k, pl.ds(i * 16, 16)] = v * mult + off if mult > 1 else v + off


def _xform_planes_1d(idx1d, off):
    for i in range(idx1d.shape[0] // 16):
        v = idx1d[pl.ds(i * 16, 16)]
        idx1d[pl.ds(i * 16, 16)] = v + off


def _spmm_body(mode, *refs):
    first = mode == "first"
    tail = mode == "tail"
    it = iter(refs)
    src_hbm, dst_hbm, tab_flat = next(it), next(it), next(it)
    if tail:
        t2_flat, rs_in, u_idx, i_idx = next(it), next(it), next(it), next(it)
        u_out, i_out, rsg_out = next(it), next(it), next(it)
        out_hbm = None
    else:
        out_hbm = next(it)
    if first:
        rs_out = next(it)
    idx_s, idx_d, rows = next(it), next(it), next(it)
    if first:
        ones, zer1, acc, acc1, sem_g, sem_s, sem_1 = it
    elif tail:
        bidx, bval, acc, sem_g, sem_s = it
    else:
        acc, sem_g, sem_s = it

    c = lax.axis_index("c")
    s = lax.axis_index("s")
    plane_off = c * N_ROWS

    def zrow(r, _):
        for j in range(HDIM // 16):
            rows[0, r, pl.ds(j * 16, 16)] = jnp.zeros((16,), jnp.float32)
        return 0

    lax.fori_loop(0, CHUNK, zrow, 0)
    if first:
        for j in range(CHUNK // 16):
            ones[pl.ds(j * 16, 16)] = jnp.ones((16,), jnp.float32)
            zer1[pl.ds(j * 16, 16)] = jnp.zeros((16,), jnp.float32)

    zsrc = rows.at[0, pl.ds(0, ZERO_CHUNK), :]
    zcps = [
        pltpu.async_copy(
            zsrc, acc.at[pl.ds(s * TILE_ACC + j * ZERO_CHUNK, ZERO_CHUNK), :],
            sem_g)
        for j in range(TILE_ACC // ZERO_CHUNK)
    ]
    if first:
        @pl.when(c == 0)
        def _():
            n_z1 = (N_ROWS // CHUNK) // NUM_TILES + jnp.where(
                s < (N_ROWS // CHUNK) % NUM_TILES, 1, 0)

            def z1body(j, _):
                pltpu.sync_copy(
                    zer1, acc1.at[pl.ds((s + j * NUM_TILES) * CHUNK, CHUNK)])
                return 0

            lax.fori_loop(0, n_z1, z1body, 0)

        @pl.when((c == 0) & (s == 0))
        def _():
            pltpu.sync_copy(zer1.at[pl.ds(0, 80)],
                            acc1.at[pl.ds(CHUNK * (N_ROWS // CHUNK), 80)])

    for z in zcps:
        z.wait()
    plsc.subcore_barrier()

    n_my = N_BLOCKS // NUM_TILES + jnp.where(s < N_BLOCKS % NUM_TILES, 1, 0)

    def ebody(j, _):
        @pl.when(j > 0)
        def _():
            for k in range(BLK):
                pltpu.make_async_copy(tab_flat.at[pl.ds(0, CHUNK), :],
                                      rows.at[k], sem_s).wait()
            if first:
                @pl.when(c == 0)
                def _():
                    for k in range(BLK):
                        pltpu.make_async_copy(rs_out.at[pl.ds(0, CHUNK)],
                                              ones, sem_1).wait()

        b = (s + j * NUM_TILES) * BLK
        pltpu.sync_copy(src_hbm.at[pl.ds(b, BLK), :], idx_s)
        pltpu.sync_copy(dst_hbm.at[pl.ds(b, BLK), :], idx_d)
        if first:
            _xform_planes(idx_s, 2, c)
        else:
            _xform_planes(idx_s, 1, plane_off)
        gathers = [
            pltpu.async_copy(tab_flat.at[idx_s.at[k]], rows.at[k], sem_g)
            for k in range(BLK)
        ]
        for k in range(BLK):
            gathers[k].wait()
            pltpu.async_copy(rows.at[k], acc.at[idx_d.at[k]], sem_s, add=True)
            if first:
                @pl.when(c == 0)
                def _():
                    pltpu.async_copy(ones, acc1.at[idx_d.at[k]], sem_1, add=True)
        return 0

    lax.fori_loop(0, n_my, ebody, 0)
    for k in range(BLK):
        pltpu.make_async_copy(tab_flat.at[pl.ds(0, CHUNK), :], rows.at[k],
                              sem_s).wait()
    if first:
        @pl.when(c == 0)
        def _():
            for k in range(BLK):
                pltpu.make_async_copy(rs_out.at[pl.ds(0, CHUNK)], ones,
                                      sem_1).wait()
    plsc.subcore_barrier()

    if not tail:
        n_wb = N_WB // NUM_TILES + jnp.where(s < N_WB % NUM_TILES, 1, 0)

        def wbody(j, _):
            r0 = (s + j * NUM_TILES) * WB_CHUNK
            pltpu.sync_copy(acc.at[pl.ds(r0, WB_CHUNK), :],
                            out_hbm.at[c, pl.ds(r0, WB_CHUNK), :])
            if first:
                @pl.when(c == 0)
                def _():
                    pltpu.sync_copy(acc1.at[pl.ds(r0, WB_CHUNK)],
                                    rs_out.at[pl.ds(r0, WB_CHUNK)])
            return 0

        lax.fori_loop(0, n_wb, wbody, 0)

    if tail:
        base = s * GB
        for half in range(GB // CHUNK):
            o = base + half * CHUNK
            pltpu.sync_copy(u_idx.at[pl.ds(o, CHUNK)], bidx)

            @pl.when(c == 0)
            def _():
                pltpu.async_copy(rs_in.at[bidx], bval, sem_g).wait()
                pltpu.sync_copy(bval, rsg_out.at[pl.ds(o, CHUNK)])

            _xform_planes_1d(bidx, plane_off)
            pltpu.async_copy(t2_flat.at[bidx], rows.at[0], sem_g).wait()
            pltpu.sync_copy(rows.at[0], u_out.at[c, pl.ds(o, CHUNK), :])

            pltpu.sync_copy(i_idx.at[pl.ds(o, CHUNK)], bidx)
            pltpu.sync_copy(acc.at[bidx], rows.at[0])
            pltpu.sync_copy(rows.at[0], i_out.at[c, pl.ds(o, CHUNK), :])


def _make_spmm(mode):
    out_type = [jax.ShapeDtypeStruct((NUM_SC, N_ROWS, HDIM), jnp.float32)]
    scratch = [
        pltpu.VMEM((BLK, CHUNK), jnp.int32),
        pltpu.VMEM((BLK, CHUNK), jnp.int32),
        pltpu.VMEM((BLK, CHUNK, HDIM), jnp.float32),
    ]
    if mode == "first":
        out_type.append(jax.ShapeDtypeStruct((N_ROWS,), jnp.float32))
        scratch += [
            pltpu.VMEM((CHUNK,), jnp.float32),
            pltpu.VMEM((CHUNK,), jnp.float32),
            pltpu.VMEM_SHARED((N_ROWS, HDIM), jnp.float32),
            pltpu.VMEM_SHARED((N_ROWS,), jnp.float32),
        ]
        scratch += [pltpu.SemaphoreType.DMA] * 3
    elif mode == "tail":
        out_type = [
            jax.ShapeDtypeStruct((NUM_SC, BATCH_SIZE, HDIM), jnp.float32),
            jax.ShapeDtypeStruct((NUM_SC, BATCH_SIZE, HDIM), jnp.float32),
            jax.ShapeDtypeStruct((BATCH_SIZE,), jnp.float32),
        ]
        scratch += [
            pltpu.VMEM((CHUNK,), jnp.int32),
            pltpu.VMEM((CHUNK,), jnp.float32),
            pltpu.VMEM_SHARED((N_ROWS, HDIM), jnp.float32),
        ]
        scratch += [pltpu.SemaphoreType.DMA] * 2
    else:
        scratch += [pltpu.VMEM_SHARED((N_ROWS, HDIM), jnp.float32)]
        scratch += [pltpu.SemaphoreType.DMA] * 2
    return pl.kernel(
        functools.partial(_spmm_body, mode),
        out_type=out_type[0] if len(out_type) == 1 else tuple(out_type),
        mesh=_SC_MESH,
        scratch_types=scratch,
        compiler_params=_SC_PARAMS,
        name="spmm_" + mode,
    )


_spmm_first = _make_spmm("first")
_spmm = _make_spmm("mid")
_spmm_tail = _make_spmm("tail")


_SCALE_BLK = 2000


def _scale_kernel(t_ref, rs_ref, o_ref):
    rs = rs_ref[...]
    inv = jnp.where(rs > 0, 1.0 / rs, 0.0)
    o_ref[...] = t_ref[...] * (inv * inv)[None]


def _scale_rows(t, rs):
    grid = N_ROWS // _SCALE_BLK
    return pl.pallas_call(
        _scale_kernel,
        grid=(grid,),
        in_specs=[
            pl.BlockSpec((NUM_SC, _SCALE_BLK, HDIM), lambda i: (0, i, 0)),
            pl.BlockSpec((_SCALE_BLK, 1), lambda i: (i, 0)),
        ],
        out_specs=pl.BlockSpec((NUM_SC, _SCALE_BLK, HDIM), lambda i: (0, i, 0)),
        out_shape=jax.ShapeDtypeStruct((NUM_SC, N_ROWS, HDIM), jnp.float32),
    )(t, rs.reshape(N_ROWS, 1))


def _score_kernel(u_ref, i_ref, rs_ref, o_ref):
    dot = jnp.sum(u_ref[0] * i_ref[0], axis=1, keepdims=True)
    dot += jnp.sum(u_ref[1] * i_ref[1], axis=1, keepdims=True)
    rs = rs_ref[...]
    inv = jnp.where(rs > 0, 1.0 / rs, 0.0)
    o_ref[...] = jax.nn.sigmoid(dot * inv)


def _score(u_rows, i_rows, rs_g):
    out = pl.pallas_call(
        _score_kernel,
        out_shape=jax.ShapeDtypeStruct((BATCH_SIZE, 1), jnp.float32),
    )(u_rows, i_rows, rs_g.reshape(BATCH_SIZE, 1))
    return out.reshape(BATCH_SIZE)


def kernel(user_indices, item_indices, user_table, item_table, adj_rows, adj_cols):
    del user_table
    src_r = adj_rows.reshape(N_IDX_ROWS, CHUNK)
    src_c = adj_cols.reshape(N_IDX_ROWS, CHUNK)

    def flat(t):
        return t.reshape(NUM_SC * N_ROWS, HDIM)

    t1, rowsum = _spmm_first(src_c, src_r, flat(item_table))
    us1 = _scale_rows(t1, rowsum)
    i1 = _spmm(src_r, src_c, flat(us1))
    t2 = _spmm(src_c, src_r, flat(i1))
    us2 = _scale_rows(t2, rowsum)
    u_rows, i_rows, rs_g = _spmm_tail(
        src_r, src_c, flat(us2), flat(t2), rowsum, user_indices, item_indices)
    return _score(u_rows, i_rows, rs_g)


# speedup vs baseline: 1.5217x; 1.1669x over previous
import jax
import jax.numpy as jnp
from jax import lax
from jax.experimental import pallas as pl
from jax.experimental.pallas import tpu as pltpu
from jax.experimental.pallas import tpu_sc as plsc

N_ROWS = 50000
N_EDGES_TOTAL = 800000
DIM = 64
HDIM = DIM // 2
BATCH_SIZE = 4096

NUM_SC = 2
NUM_TILES = 16
CHUNK = 128
BLK = 5
N_IDX_ROWS = N_EDGES_TOTAL // CHUNK
N_BLOCKS = N_IDX_ROWS // BLK
WB = 125
N_WB_TILE = (N_ROWS // WB) // NUM_TILES
RS_ROWS = 50048

_SC_MESH = plsc.VectorSubcoreMesh(core_axis_name="c", subcore_axis_name="s")
_SC_PARAMS = pltpu.CompilerParams(use_tc_tiling_on_sc=False,
                                  needs_layout_passes=False)


def _lightgcn_body(rows2d, cols2d, item_flat, u_idx, i_idx,
                   tmp1, tmp2, t2us, t2raw, u_out, i_out, rsg_out,
                   idx_s, idx_d, rows, zbuf, zer1, ones, invv, bidx, bval,
                   acc, acc1, sem_g, sem_s, sem_1):
    c = lax.axis_index("c")
    s = lax.axis_index("s")

    def zrow(r, _):
        for h in range(HDIM // 16):
            zbuf[r, pl.ds(h * 16, 16)] = jnp.zeros((16,), jnp.float32)
        return 0

    lax.fori_loop(0, CHUNK, zrow, 0)
    for i in range(CHUNK // 16):
        ones[pl.ds(i * 16, 16)] = jnp.ones((16,), jnp.float32)
        zer1[pl.ds(i * 16, 16)] = jnp.zeros((16,), jnp.float32)

    zcps = [
        pltpu.async_copy(zbuf.at[pl.ds(0, WB), :],
                         acc.at[pl.ds(s * (N_ROWS // NUM_TILES) + j * WB, WB), :],
                         sem_g)
        for j in range(N_WB_TILE)
    ]
    n_z1 = (RS_ROWS // CHUNK) // NUM_TILES + jnp.where(
        s < (RS_ROWS // CHUNK) % NUM_TILES, 1, 0)

    def z1body(j, _):
        pltpu.sync_copy(zer1, acc1.at[pl.ds((s + j * NUM_TILES) * CHUNK, CHUNK)])
        return 0

    lax.fori_loop(0, n_z1, z1body, 0)
    for z in zcps:
        z.wait()
    plsc.subcore_barrier()

    n_my = N_BLOCKS // NUM_TILES + jnp.where(s < N_BLOCKS % NUM_TILES, 1, 0)

    def edge_phase(gather_ref, src2d, dst2d, do_ones, xform):
        def drain():
            for k in range(BLK):
                pltpu.make_async_copy(item_flat.at[pl.ds(0, CHUNK), :],
                                      rows.at[k], sem_s).wait()
            if do_ones:
                for k in range(BLK):
                    pltpu.make_async_copy(rsg_out.at[pl.ds(0, CHUNK)],
                                          ones, sem_1).wait()

        def ebody(j, _):
            @pl.when(j > 0)
            def _():
                drain()

            b = (s + j * NUM_TILES) * BLK
            pltpu.sync_copy(src2d.at[pl.ds(b, BLK), :], idx_s)
            pltpu.sync_copy(dst2d.at[pl.ds(b, BLK), :], idx_d)
            if xform:
                for k in range(BLK):
                    for i in range(CHUNK // 16):
                        v = idx_s[k, pl.ds(i * 16, 16)]
                        idx_s[k, pl.ds(i * 16, 16)] = v * 2 + c
            gathers = [
                pltpu.async_copy(gather_ref(k), rows.at[k], sem_g)
                for k in range(BLK)
            ]
            for k in range(BLK):
                gathers[k].wait()
                pltpu.async_copy(rows.at[k], acc.at[idx_d.at[k]], sem_s,
                                 add=True)
                if do_ones:
                    pltpu.async_copy(ones, acc1.at[idx_d.at[k]], sem_1,
                                     add=True)
            return 0

        lax.fori_loop(0, n_my, ebody, 0)
        drain()
        plsc.subcore_barrier()

    def load_inv2(r0):
        for i in range(CHUNK // 16):
            bidx[pl.ds(i * 16, 16)] = lax.iota(jnp.int32, 16) + (r0 + i * 16)
        pltpu.async_copy(acc1.at[bidx], invv, sem_g).wait()
        for i in range(CHUNK // 16):
            v = invv[pl.ds(i * 16, 16)]
            iv = jnp.where(v > 0, 1.0 / v, 0.0)
            invv[pl.ds(i * 16, 16)] = iv * iv

    def writeback(raw_out, scaled_out):
        def wbody(j, _):
            r0 = (s + j * NUM_TILES) * WB
            if scaled_out is not None:
                pltpu.sync_copy(acc.at[pl.ds(r0, WB), :],
                                rows.at[0, pl.ds(0, WB), :])
                if raw_out is not None:
                    pltpu.sync_copy(rows.at[0, pl.ds(0, WB), :],
                                    raw_out.at[c, pl.ds(r0, WB), :])
                load_inv2(r0)

                def sbody(r, _):
                    m = plsc.load_gather(invv, [jnp.full((16,), r, jnp.int32)])
                    for h in range(HDIM // 16):
                        rows[0, r, pl.ds(h * 16, 16)] = (
                            rows[0, r, pl.ds(h * 16, 16)] * m)
                    return 0

                lax.fori_loop(0, WB, sbody, 0)
                pltpu.sync_copy(rows.at[0, pl.ds(0, WB), :],
                                scaled_out.at[c, pl.ds(r0, WB), :])
            else:
                pltpu.sync_copy(acc.at[pl.ds(r0, WB), :],
                                raw_out.at[c, pl.ds(r0, WB), :])
            pltpu.sync_copy(zbuf.at[pl.ds(0, WB), :], acc.at[pl.ds(r0, WB), :])
            return 0

        lax.fori_loop(0, N_WB_TILE, wbody, 0)
        plsc.subcore_barrier()

    edge_phase(lambda k: item_flat.at[idx_s.at[k]], cols2d, rows2d, True, True)
    writeback(None, tmp1)
    edge_phase(lambda k: tmp1.at[c].at[idx_s.at[k]], rows2d, cols2d, False, False)
    writeback(tmp2, None)
    edge_phase(lambda k: tmp2.at[c].at[idx_s.at[k]], cols2d, rows2d, False, False)
    writeback(t2raw, t2us)
    edge_phase(lambda k: t2us.at[c].at[idx_s.at[k]], rows2d, cols2d, False, False)

    base = s * (BATCH_SIZE // NUM_TILES)
    for half in range(BATCH_SIZE // NUM_TILES // CHUNK):
        o = base + half * CHUNK
        pltpu.sync_copy(u_idx.at[pl.ds(o, CHUNK)], bidx)

        @pl.when(c == 0)
        def _():
            pltpu.async_copy(acc1.at[bidx], bval, sem_g).wait()
            pltpu.sync_copy(bval, rsg_out.at[pl.ds(o, CHUNK)])

        pltpu.async_copy(t2raw.at[c].at[bidx], rows.at[0], sem_g).wait()
        pltpu.sync_copy(rows.at[0], u_out.at[c, pl.ds(o, CHUNK), :])

        pltpu.sync_copy(i_idx.at[pl.ds(o, CHUNK)], bidx)
        pltpu.sync_copy(acc.at[bidx], rows.at[0])
        pltpu.sync_copy(rows.at[0], i_out.at[c, pl.ds(o, CHUNK), :])


_lightgcn = pl.kernel(
    _lightgcn_body,
    out_type=(
        jax.ShapeDtypeStruct((NUM_SC, N_ROWS, HDIM), jnp.float32),
        jax.ShapeDtypeStruct((NUM_SC, N_ROWS, HDIM), jnp.float32),
        jax.ShapeDtypeStruct((NUM_SC, N_ROWS, HDIM), jnp.float32),
        jax.ShapeDtypeStruct((NUM_SC, N_ROWS, HDIM), jnp.float32),
        jax.ShapeDtypeStruct((NUM_SC, BATCH_SIZE, HDIM), jnp.float32),
        jax.ShapeDtypeStruct((NUM_SC, BATCH_SIZE, HDIM), jnp.float32),
        jax.ShapeDtypeStruct((BATCH_SIZE,), jnp.float32),
    ),
    mesh=_SC_MESH,
    scratch_types=[
        pltpu.VMEM((BLK, CHUNK), jnp.int32),
        pltpu.VMEM((BLK, CHUNK), jnp.int32),
        pltpu.VMEM((BLK, CHUNK, HDIM), jnp.float32),
        pltpu.VMEM((CHUNK, HDIM), jnp.float32),
        pltpu.VMEM((CHUNK,), jnp.float32),
        pltpu.VMEM((CHUNK,), jnp.float32),
        pltpu.VMEM((CHUNK,), jnp.float32),
        pltpu.VMEM((CHUNK,), jnp.int32),
        pltpu.VMEM((CHUNK,), jnp.float32),
        pltpu.VMEM_SHARED((N_ROWS, HDIM), jnp.float32),
        pltpu.VMEM_SHARED((RS_ROWS,), jnp.float32),
        pltpu.SemaphoreType.DMA,
        pltpu.SemaphoreType.DMA,
        pltpu.SemaphoreType.DMA,
    ],
    compiler_params=_SC_PARAMS,
    name="lightgcn_sc",
)


def _score_kernel(u_ref, i_ref, rs_ref, o_ref):
    dot = jnp.sum(u_ref[0] * i_ref[0], axis=1, keepdims=True)
    dot += jnp.sum(u_ref[1] * i_ref[1], axis=1, keepdims=True)
    rs = rs_ref[...]
    inv = jnp.where(rs > 0, 1.0 / rs, 0.0)
    o_ref[...] = jax.nn.sigmoid(dot * inv)


def _score(u_rows, i_rows, rs_g):
    out = pl.pallas_call(
        _score_kernel,
        out_shape=jax.ShapeDtypeStruct((BATCH_SIZE, 1), jnp.float32),
    )(u_rows, i_rows, rs_g.reshape(BATCH_SIZE, 1))
    return out.reshape(BATCH_SIZE)


def kernel(user_indices, item_indices, user_table, item_table, adj_rows, adj_cols):
    del user_table
    rows2d = adj_rows.reshape(N_IDX_ROWS, CHUNK)
    cols2d = adj_cols.reshape(N_IDX_ROWS, CHUNK)
    item_flat = item_table.reshape(NUM_SC * N_ROWS, HDIM)
    _, _, _, _, u_rows, i_rows, rs_g = _lightgcn(
        rows2d, cols2d, item_flat, user_indices, item_indices)
    return _score(u_rows, i_rows, rs_g)


# speedup vs baseline: 2.1289x; 1.3990x over previous
import jax
import jax.numpy as jnp
from jax import lax
from jax.experimental import pallas as pl
from jax.experimental.pallas import tpu as pltpu
from jax.experimental.pallas import tpu_sc as plsc

N_ROWS = 50000
N_EDGES_TOTAL = 800000
DIM = 64
HDIM = DIM // 2
BATCH_SIZE = 4096

NUM_SC = 2
NUM_TILES = 16
CHUNK = 128
BLK = 5
N_IDX_ROWS = N_EDGES_TOTAL // CHUNK
N_BLOCKS = N_IDX_ROWS // BLK
WB = 125
N_WB_TILE = (N_ROWS // WB) // NUM_TILES
RS_ROWS = 50048

_SC_MESH = plsc.VectorSubcoreMesh(core_axis_name="c", subcore_axis_name="s")
_SC_PARAMS = pltpu.CompilerParams(use_tc_tiling_on_sc=False,
                                  needs_layout_passes=False)


def _lightgcn_body(rows2d, cols2d, item_flat, u_idx, i_idx,
                   tmp1, tmp2, t2us, t2raw, u_out, i_out, rsg_out,
                   idx_s, idx_d, rows, zbuf, zer1, ones, invv, bidx, bval,
                   acc, acc1, sem_g, sem_s, sem_1, sem_i):
    c = lax.axis_index("c")
    s = lax.axis_index("s")

    def zrow(r, _):
        for h in range(HDIM // 16):
            zbuf[r, pl.ds(h * 16, 16)] = jnp.zeros((16,), jnp.float32)
        return 0

    lax.fori_loop(0, CHUNK, zrow, 0)
    for i in range(CHUNK // 16):
        ones[pl.ds(i * 16, 16)] = jnp.ones((16,), jnp.float32)
        zer1[pl.ds(i * 16, 16)] = jnp.zeros((16,), jnp.float32)

    zcps = [
        pltpu.async_copy(zbuf.at[pl.ds(0, WB), :],
                         acc.at[pl.ds(s * (N_ROWS // NUM_TILES) + j * WB, WB), :],
                         sem_g)
        for j in range(N_WB_TILE)
    ]
    n_z1 = (RS_ROWS // CHUNK) // NUM_TILES + jnp.where(
        s < (RS_ROWS // CHUNK) % NUM_TILES, 1, 0)

    def z1body(j, _):
        pltpu.sync_copy(zer1, acc1.at[pl.ds((s + j * NUM_TILES) * CHUNK, CHUNK)])
        return 0

    lax.fori_loop(0, n_z1, z1body, 0)
    for z in zcps:
        z.wait()
    plsc.subcore_barrier()

    n_my = N_BLOCKS // NUM_TILES + jnp.where(s < N_BLOCKS % NUM_TILES, 1, 0)

    def edge_phase(gather_ref, src2d, dst2d, do_ones, xform):
        def drain_scatters():
            for k in range(BLK):
                pltpu.make_async_copy(item_flat.at[pl.ds(0, CHUNK), :],
                                      rows.at[k], sem_s).wait()
            if do_ones:
                for k in range(BLK):
                    pltpu.make_async_copy(rsg_out.at[pl.ds(0, CHUNK)],
                                          ones, sem_1).wait()

        def issue_idx(jb, p):
            b = (s + jb * NUM_TILES) * BLK
            pltpu.async_copy(src2d.at[pl.ds(b, BLK), :], idx_s.at[p], sem_i)
            pltpu.async_copy(dst2d.at[pl.ds(b, BLK), :], idx_d.at[p], sem_i)

        def wait_idx(p):
            pltpu.make_async_copy(src2d.at[pl.ds(0, BLK), :], idx_s.at[p],
                                  sem_i).wait()
            pltpu.make_async_copy(src2d.at[pl.ds(0, BLK), :], idx_d.at[p],
                                  sem_i).wait()

        def sub_block(jb, p):
            @pl.when(jb < n_my)
            def _():
                @pl.when(jb > 0)
                def _():
                    drain_scatters()

                @pl.when(jb + 1 < n_my)
                def _():
                    issue_idx(jb + 1, 1 - p)

                wait_idx(p)
                if xform:
                    for k in range(BLK):
                        for i in range(CHUNK // 16):
                            v = idx_s[p, k, pl.ds(i * 16, 16)]
                            idx_s[p, k, pl.ds(i * 16, 16)] = v * 2 + c
                gathers = [
                    pltpu.async_copy(gather_ref(p, k), rows.at[k], sem_g)
                    for k in range(BLK)
                ]
                for k in range(BLK):
                    gathers[k].wait()
                    pltpu.async_copy(rows.at[k], acc.at[idx_d.at[p, k]],
                                     sem_s, add=True)
                    if do_ones:
                        pltpu.async_copy(ones, acc1.at[idx_d.at[p, k]],
                                         sem_1, add=True)

        issue_idx(0, 0)

        def ebody(jj, _):
            sub_block(jj * 2, 0)
            sub_block(jj * 2 + 1, 1)
            return 0

        lax.fori_loop(0, (n_my + 1) // 2, ebody, 0)
        drain_scatters()
        plsc.subcore_barrier()

    def load_inv2(r0):
        for i in range(CHUNK // 16):
            bidx[pl.ds(i * 16, 16)] = lax.iota(jnp.int32, 16) + (r0 + i * 16)
        pltpu.async_copy(acc1.at[bidx], invv, sem_g).wait()
        for i in range(CHUNK // 16):
            v = invv[pl.ds(i * 16, 16)]
            iv = jnp.where(v > 0, 1.0 / v, 0.0)
            invv[pl.ds(i * 16, 16)] = iv * iv

    def writeback(raw_out, scaled_out):
        def wbody(j, _):
            r0 = (s + j * NUM_TILES) * WB
            if scaled_out is not None:
                pltpu.sync_copy(acc.at[pl.ds(r0, WB), :],
                                rows.at[0, pl.ds(0, WB), :])
                if raw_out is not None:
                    pltpu.sync_copy(rows.at[0, pl.ds(0, WB), :],
                                    raw_out.at[c, pl.ds(r0, WB), :])
                load_inv2(r0)

                def sbody(r, _):
                    m = plsc.load_gather(invv, [jnp.full((16,), r, jnp.int32)])
                    for h in range(HDIM // 16):
                        rows[0, r, pl.ds(h * 16, 16)] = (
                            rows[0, r, pl.ds(h * 16, 16)] * m)
                    return 0

                lax.fori_loop(0, WB, sbody, 0)
                pltpu.sync_copy(rows.at[0, pl.ds(0, WB), :],
                                scaled_out.at[c, pl.ds(r0, WB), :])
            else:
                pltpu.sync_copy(acc.at[pl.ds(r0, WB), :],
                                raw_out.at[c, pl.ds(r0, WB), :])
            pltpu.sync_copy(zbuf.at[pl.ds(0, WB), :], acc.at[pl.ds(r0, WB), :])
            return 0

        lax.fori_loop(0, N_WB_TILE, wbody, 0)
        plsc.subcore_barrier()

    edge_phase(lambda p, k: item_flat.at[idx_s.at[p, k]], cols2d, rows2d,
               True, True)
    writeback(None, tmp1)
    edge_phase(lambda p, k: tmp1.at[c].at[idx_s.at[p, k]], rows2d, cols2d,
               False, False)
    writeback(tmp2, None)
    edge_phase(lambda p, k: tmp2.at[c].at[idx_s.at[p, k]], cols2d, rows2d,
               False, False)
    writeback(t2raw, t2us)
    edge_phase(lambda p, k: t2us.at[c].at[idx_s.at[p, k]], rows2d, cols2d,
               False, False)

    base = s * (BATCH_SIZE // NUM_TILES)
    for half in range(BATCH_SIZE // NUM_TILES // CHUNK):
        o = base + half * CHUNK
        pltpu.sync_copy(u_idx.at[pl.ds(o, CHUNK)], bidx)

        @pl.when(c == 0)
        def _():
            pltpu.async_copy(acc1.at[bidx], bval, sem_g).wait()
            pltpu.sync_copy(bval, rsg_out.at[pl.ds(o, CHUNK)])

        pltpu.async_copy(t2raw.at[c].at[bidx], rows.at[0], sem_g).wait()
        pltpu.sync_copy(rows.at[0], u_out.at[c, pl.ds(o, CHUNK), :])

        pltpu.sync_copy(i_idx.at[pl.ds(o, CHUNK)], bidx)
        pltpu.sync_copy(acc.at[bidx], rows.at[0])
        pltpu.sync_copy(rows.at[0], i_out.at[c, pl.ds(o, CHUNK), :])


_lightgcn = pl.kernel(
    _lightgcn_body,
    out_type=(
        jax.ShapeDtypeStruct((NUM_SC, N_ROWS, HDIM), jnp.float32),
        jax.ShapeDtypeStruct((NUM_SC, N_ROWS, HDIM), jnp.float32),
        jax.ShapeDtypeStruct((NUM_SC, N_ROWS, HDIM), jnp.float32),
        jax.ShapeDtypeStruct((NUM_SC, N_ROWS, HDIM), jnp.float32),
        jax.ShapeDtypeStruct((NUM_SC, BATCH_SIZE, HDIM), jnp.float32),
        jax.ShapeDtypeStruct((NUM_SC, BATCH_SIZE, HDIM), jnp.float32),
        jax.ShapeDtypeStruct((BATCH_SIZE,), jnp.float32),
    ),
    mesh=_SC_MESH,
    scratch_types=[
        pltpu.VMEM((2, BLK, CHUNK), jnp.int32),
        pltpu.VMEM((2, BLK, CHUNK), jnp.int32),
        pltpu.VMEM((BLK, CHUNK, HDIM), jnp.float32),
        pltpu.VMEM((CHUNK, HDIM), jnp.float32),
        pltpu.VMEM((CHUNK,), jnp.float32),
        pltpu.VMEM((CHUNK,), jnp.float32),
        pltpu.VMEM((CHUNK,), jnp.float32),
        pltpu.VMEM((CHUNK,), jnp.int32),
        pltpu.VMEM((CHUNK,), jnp.float32),
        pltpu.VMEM_SHARED((N_ROWS, HDIM), jnp.float32),
        pltpu.VMEM_SHARED((RS_ROWS,), jnp.float32),
        pltpu.SemaphoreType.DMA,
        pltpu.SemaphoreType.DMA,
        pltpu.SemaphoreType.DMA,
        pltpu.SemaphoreType.DMA,
    ],
    compiler_params=_SC_PARAMS,
    name="lightgcn_sc",
)


def _score_kernel(u_ref, i_ref, rs_ref, o_ref):
    dot = jnp.sum(u_ref[0] * i_ref[0], axis=1, keepdims=True)
    dot += jnp.sum(u_ref[1] * i_ref[1], axis=1, keepdims=True)
    rs = rs_ref[...]
    inv = jnp.where(rs > 0, 1.0 / rs, 0.0)
    o_ref[...] = jax.nn.sigmoid(dot * inv)


def _score(u_rows, i_rows, rs_g):
    out = pl.pallas_call(
        _score_kernel,
        out_shape=jax.ShapeDtypeStruct((BATCH_SIZE, 1), jnp.float32),
    )(u_rows, i_rows, rs_g.reshape(BATCH_SIZE, 1))
    return out.reshape(BATCH_SIZE)


def kernel(user_indices, item_indices, user_table, item_table, adj_rows, adj_cols):
    del user_table
    rows2d = adj_rows.reshape(N_IDX_ROWS, CHUNK)
    cols2d = adj_cols.reshape(N_IDX_ROWS, CHUNK)
    item_flat = item_table.reshape(NUM_SC * N_ROWS, HDIM)
    _, _, _, _, u_rows, i_rows, rs_g = _lightgcn(
        rows2d, cols2d, item_flat, user_indices, item_indices)
    return _score(u_rows, i_rows, rs_g)
